# baseline (speedup 1.0000x reference)
import functools

import jax
import jax.numpy as jnp
from jax import lax
from jax.experimental import pallas as pl
from jax.experimental.pallas import tpu as pltpu
from jax.experimental.pallas import tpu_sc as plsc

_NUM_CORES = 2
_NUM_SUBCORES = 16
_LANES = 16


def _gather_mean_sc(x_t, emb):
    S, B = x_t.shape
    V, E = emb.shape
    NW = _NUM_CORES * _NUM_SUBCORES
    EPW = B // NW
    mesh = plsc.VectorSubcoreMesh(
        core_axis_name="c", subcore_axis_name="s",
        num_cores=_NUM_CORES, num_subcores=_NUM_SUBCORES)

    @functools.partial(
        pl.kernel,
        out_type=jax.ShapeDtypeStruct((B, E), jnp.float32),
        mesh=mesh,
        scratch_types=[
            pltpu.VMEM((S, EPW), jnp.int32),
            pltpu.VMEM((EPW, E), jnp.float32),
            pltpu.VMEM((EPW, E), jnp.float32),
            pltpu.VMEM((EPW, E), jnp.float32),
            pltpu.SemaphoreType.DMA,
            pltpu.SemaphoreType.DMA,
        ],
    )
    def k(emb_hbm, xt_hbm, out_hbm, idx_v, rows0_v, rows1_v, acc_v,
          sem0, sem1):
        wid = lax.axis_index("c") * _NUM_SUBCORES + lax.axis_index("s")
        base = wid * EPW
        pltpu.sync_copy(xt_hbm.at[:, pl.ds(base, EPW)], idx_v)
        bufs = (rows0_v, rows1_v)
        sems = (sem0, sem1)
        copies = [None, None]
        copies[0] = pltpu.async_copy(emb_hbm.at[idx_v.at[0]], bufs[0], sem0)
        for s in range(S):
            if s + 1 < S:
                nxt = (s + 1) % 2
                copies[nxt] = pltpu.async_copy(
                    emb_hbm.at[idx_v.at[s + 1]], bufs[nxt], sems[nxt])
            copies[s % 2].wait()
            buf = bufs[s % 2]

            def body(r, _, buf=buf, s=s):
                for j in range(E // _LANES):
                    sl = pl.ds(j * _LANES, _LANES)
                    v = buf[r, sl]
                    if s == 0:
                        acc_v[r, sl] = v
                    elif s == S - 1:
                        acc_v[r, sl] = (acc_v[r, sl] + v) * (1.0 / S)
                    else:
                        acc_v[r, sl] = acc_v[r, sl] + v
                return 0

            lax.fori_loop(0, EPW, body, 0)
        pltpu.sync_copy(acc_v, out_hbm.at[pl.ds(base, EPW), :])

    return k(emb, x_t)


def _mm_body(h_ref, w_ref, b_ref, o_ref):
    o_ref[...] = lax.dot_general(
        h_ref[...], w_ref[...],
        dimension_numbers=(((1,), (1,)), ((), ())),
        preferred_element_type=jnp.float32) + b_ref[...]


def _project_tc(h, W, b2d, bn=512):
    Bm, E = h.shape
    V = W.shape[0]
    return pl.pallas_call(
        _mm_body,
        grid=(pl.cdiv(V, bn),),
        in_specs=[
            pl.BlockSpec((Bm, E), lambda j: (0, 0)),
            pl.BlockSpec((bn, E), lambda j: (j, 0)),
            pl.BlockSpec((1, bn), lambda j: (0, j)),
        ],
        out_specs=pl.BlockSpec((Bm, bn), lambda j: (0, j)),
        out_shape=jax.ShapeDtypeStruct((Bm, V), jnp.float32),
        compiler_params=pltpu.CompilerParams(
            dimension_semantics=("arbitrary",)),
    )(h, W, b2d)


def kernel(x, emb, W, b):
    x_t = x.T.astype(jnp.int32)
    h = _gather_mean_sc(x_t, emb)
    return _project_tc(h, W, b.reshape(1, -1))


# speedup vs baseline: 1.0038x; 1.0038x over previous
import functools

import jax
import jax.numpy as jnp
from jax import lax
from jax.experimental import pallas as pl
from jax.experimental.pallas import tpu as pltpu
from jax.experimental.pallas import tpu_sc as plsc

_NUM_CORES = 2
_NUM_SUBCORES = 16
_LANES = 16


def _gather_mean_sc(x_t, emb):
    S, B = x_t.shape
    V, E = emb.shape
    NW = _NUM_CORES * _NUM_SUBCORES
    EPW = B // NW
    mesh = plsc.VectorSubcoreMesh(
        core_axis_name="c", subcore_axis_name="s",
        num_cores=_NUM_CORES, num_subcores=_NUM_SUBCORES)

    @functools.partial(
        pl.kernel,
        out_type=jax.ShapeDtypeStruct((B, E), jnp.float32),
        mesh=mesh,
        scratch_types=[
            pltpu.VMEM((S, EPW), jnp.int32),
            pltpu.VMEM((EPW, E), jnp.float32),
            pltpu.VMEM((EPW, E), jnp.float32),
            pltpu.VMEM((EPW, E), jnp.float32),
            pltpu.SemaphoreType.DMA,
            pltpu.SemaphoreType.DMA,
        ],
    )
    def k(emb_hbm, xt_hbm, out_hbm, idx_v, rows0_v, rows1_v, acc_v,
          sem0, sem1):
        wid = lax.axis_index("c") * _NUM_SUBCORES + lax.axis_index("s")
        base = wid * EPW
        pltpu.sync_copy(xt_hbm.at[:, pl.ds(base, EPW)], idx_v)
        bufs = (rows0_v, rows1_v)
        sems = (sem0, sem1)
        copies = [None, None]
        copies[0] = pltpu.async_copy(emb_hbm.at[idx_v.at[0]], bufs[0], sem0)
        for s in range(S):
            if s + 1 < S:
                nxt = (s + 1) % 2
                copies[nxt] = pltpu.async_copy(
                    emb_hbm.at[idx_v.at[s + 1]], bufs[nxt], sems[nxt])
            copies[s % 2].wait()
            buf = bufs[s % 2]

            def body(r, _, buf=buf, s=s):
                for j in range(E // _LANES):
                    sl = pl.ds(j * _LANES, _LANES)
                    v = buf[r, sl]
                    if s == 0:
                        acc_v[r, sl] = v
                    elif s == S - 1:
                        acc_v[r, sl] = (acc_v[r, sl] + v) * (1.0 / S)
                    else:
                        acc_v[r, sl] = acc_v[r, sl] + v
                return 0

            lax.fori_loop(0, EPW, body, 0)
        pltpu.sync_copy(acc_v, out_hbm.at[pl.ds(base, EPW), :])

    return k(emb, x_t)


def _mm_body(h_ref, w_ref, b_ref, o_ref):
    o_ref[...] = lax.dot_general(
        h_ref[...], w_ref[...],
        dimension_numbers=(((1,), (1,)), ((), ())),
        preferred_element_type=jnp.float32) + b_ref[...]


def _project_tc(h, W, b2d, bn=1024):
    Bm, E = h.shape
    V = W.shape[0]
    return pl.pallas_call(
        _mm_body,
        grid=(pl.cdiv(V, bn),),
        in_specs=[
            pl.BlockSpec((Bm, E), lambda j: (0, 0)),
            pl.BlockSpec((bn, E), lambda j: (j, 0)),
            pl.BlockSpec((1, bn), lambda j: (0, j)),
        ],
        out_specs=pl.BlockSpec((Bm, bn), lambda j: (0, j)),
        out_shape=jax.ShapeDtypeStruct((Bm, V), jnp.float32),
        compiler_params=pltpu.CompilerParams(
            dimension_semantics=("arbitrary",)),
    )(h, W, b2d)


def kernel(x, emb, W, b):
    x_t = x.T.astype(jnp.int32)
    h = _gather_mean_sc(x_t, emb)
    return _project_tc(h, W, b.reshape(1, -1))
